# baseline (speedup 1.0000x reference)
import jax
import jax.numpy as jnp
from jax.experimental import pallas as pl
from jax.experimental.pallas import tpu as pltpu

DIM = 1024
INTER = 512
N_EXPERTS = 8
N_LOCAL = 4
N_AUG = 6
T = 2048
BT = 256


def _moe_body(x_ref, gate_ref, w1_ref, w3_ref, w2_ref, b1_ref, b3_ref,
              b2_ref, out_ref):
    j = pl.program_id(1)
    xb = x_ref[...]

    logits = jax.lax.dot_general(
        xb, gate_ref[...], (((1,), (1,)), ((), ())),
        preferred_element_type=jnp.float32)
    m = jnp.max(logits, axis=1, keepdims=True)
    ex = jnp.exp(logits - m)
    scores = ex / jnp.sum(ex, axis=1, keepdims=True)

    iota8 = jax.lax.broadcasted_iota(jnp.int32, (BT, N_EXPERTS), 1)
    m0 = jnp.max(scores, axis=1, keepdims=True)
    idx0 = jnp.min(jnp.where(scores == m0, iota8, N_EXPERTS), axis=1,
                   keepdims=True)
    s1 = jnp.where(iota8 == idx0, -jnp.inf, scores)
    m1 = jnp.max(s1, axis=1, keepdims=True)
    idx1 = jnp.min(jnp.where(s1 == m1, iota8, N_EXPERTS), axis=1,
                   keepdims=True)

    gw_routed = (jnp.where(idx0 == j, m0, 0.0)
                 + jnp.where(idx1 == j, m1, 0.0))
    gw = jnp.where(j < N_LOCAL, gw_routed, 1.0)

    w1 = w1_ref[0]
    w3 = w3_ref[0]
    w2 = w2_ref[0]
    h1 = jax.lax.dot_general(xb, w1, (((1,), (1,)), ((), ())),
                             preferred_element_type=jnp.float32) + b1_ref[0]
    h3 = jax.lax.dot_general(xb, w3, (((1,), (1,)), ((), ())),
                             preferred_element_type=jnp.float32) + b3_ref[0]
    h = (h1 * jax.lax.logistic(h1)) * h3
    o = jax.lax.dot_general(h, w2, (((1,), (1,)), ((), ())),
                            preferred_element_type=jnp.float32) + b2_ref[0]
    contrib = gw * o

    @pl.when(j == 0)
    def _():
        out_ref[...] = contrib

    @pl.when(j > 0)
    def _():
        out_ref[...] = out_ref[...] + contrib


@jax.jit
def _moe_fused(x, gate_w, w1a, w3a, w2a, b1a, b3a, b2a):
    grid = (T // BT, N_AUG)
    return pl.pallas_call(
        _moe_body,
        grid=grid,
        in_specs=[
            pl.BlockSpec((BT, DIM), lambda i, j: (i, 0)),
            pl.BlockSpec((N_EXPERTS, DIM), lambda i, j: (0, 0)),
            pl.BlockSpec((1, INTER, DIM), lambda i, j: (j, 0, 0)),
            pl.BlockSpec((1, INTER, DIM), lambda i, j: (j, 0, 0)),
            pl.BlockSpec((1, DIM, INTER), lambda i, j: (j, 0, 0)),
            pl.BlockSpec((1, 1, INTER), lambda i, j: (j, 0, 0)),
            pl.BlockSpec((1, 1, INTER), lambda i, j: (j, 0, 0)),
            pl.BlockSpec((1, 1, DIM), lambda i, j: (j, 0, 0)),
        ],
        out_specs=pl.BlockSpec((BT, DIM), lambda i, j: (i, 0)),
        out_shape=jax.ShapeDtypeStruct((T, DIM), jnp.float32),
        compiler_params=pltpu.CompilerParams(
            dimension_semantics=("arbitrary", "arbitrary")),
    )(x, gate_w, w1a, w3a, w2a, b1a, b3a, b2a)


def kernel(x, gate_w, w1, b1, w2, b2, w3, b3, ws1, bs1, ws2, bs2):
    n_shared = ws1.shape[0] // INTER
    w1a = jnp.concatenate([w1, ws1.reshape(n_shared, INTER, DIM)], axis=0)
    w3a = jnp.concatenate([w3, jnp.zeros((n_shared, INTER, DIM), w3.dtype)],
                          axis=0)
    w2a = jnp.concatenate(
        [w2, jnp.moveaxis(ws2.reshape(DIM, n_shared, INTER), 1, 0)], axis=0)
    b1a = jnp.concatenate([b1, bs1.reshape(n_shared, INTER)], axis=0)
    b3a = jnp.concatenate([b3, jnp.ones((n_shared, INTER), b3.dtype)], axis=0)
    b2a = jnp.concatenate(
        [b2, jnp.stack([bs2, jnp.zeros_like(bs2)], axis=0)], axis=0)
    return _moe_fused(x, gate_w, w1a, w3a, w2a,
                      b1a[:, None, :], b3a[:, None, :], b2a[:, None, :])


# speedup vs baseline: 1.2114x; 1.2114x over previous
import jax
import jax.numpy as jnp
from jax.experimental import pallas as pl
from jax.experimental.pallas import tpu as pltpu

DIM = 1024
INTER = 512
N_EXPERTS = 8
N_LOCAL = 4
N_AUG = 6
T = 2048
BT = 256


def _moe_body(x_ref, gate_ref, w1_ref, w3_ref, w2_ref, b1_ref, b3_ref,
              b2_ref, out_ref, gw_scr, xb_scr):
    j = pl.program_id(1)

    @pl.when(j == 0)
    def _():
        xb32 = x_ref[...]
        xb_scr[...] = xb32.astype(jnp.bfloat16)
        logits = jax.lax.dot_general(
            xb32, gate_ref[...], (((1,), (1,)), ((), ())),
            preferred_element_type=jnp.float32)
        mx = jnp.max(logits, axis=1, keepdims=True)
        ex = jnp.exp(logits - mx)
        scores = ex / jnp.sum(ex, axis=1, keepdims=True)

        iota8 = jax.lax.broadcasted_iota(jnp.int32, (BT, N_EXPERTS), 1)
        m0 = jnp.max(scores, axis=1, keepdims=True)
        idx0 = jnp.min(jnp.where(scores == m0, iota8, N_EXPERTS), axis=1,
                       keepdims=True)
        s1 = jnp.where(iota8 == idx0, -jnp.inf, scores)
        m1 = jnp.max(s1, axis=1, keepdims=True)
        idx1 = jnp.min(jnp.where(s1 == m1, iota8, N_EXPERTS), axis=1,
                       keepdims=True)
        gw_scr[...] = (jnp.where(idx0 == iota8, m0, 0.0)
                       + jnp.where(idx1 == iota8, m1, 0.0))

    xb = xb_scr[...]
    gw_routed = jnp.sum(
        jnp.where(jax.lax.broadcasted_iota(jnp.int32, (BT, N_EXPERTS), 1)
                  == j, gw_scr[...], 0.0),
        axis=1, keepdims=True)
    gw = jnp.where(j < N_LOCAL, gw_routed, 1.0)

    w1 = w1_ref[0]
    w3 = w3_ref[0]
    w2 = w2_ref[0]
    h1 = jax.lax.dot_general(xb, w1, (((1,), (1,)), ((), ())),
                             preferred_element_type=jnp.float32) + b1_ref[0]
    h3 = jax.lax.dot_general(xb, w3, (((1,), (1,)), ((), ())),
                             preferred_element_type=jnp.float32) + b3_ref[0]
    h = (h1 * jax.lax.logistic(h1)) * h3
    o = jax.lax.dot_general(h.astype(jnp.bfloat16), w2,
                            (((1,), (1,)), ((), ())),
                            preferred_element_type=jnp.float32) + b2_ref[0]
    contrib = gw * o

    @pl.when(j == 0)
    def _():
        out_ref[...] = contrib

    @pl.when(j > 0)
    def _():
        out_ref[...] = out_ref[...] + contrib


@jax.jit
def _moe_fused(x, gate_w, w1a, w3a, w2a, b1a, b3a, b2a):
    grid = (T // BT, N_AUG)
    return pl.pallas_call(
        _moe_body,
        grid=grid,
        in_specs=[
            pl.BlockSpec((BT, DIM), lambda i, j: (i, 0)),
            pl.BlockSpec((N_EXPERTS, DIM), lambda i, j: (0, 0)),
            pl.BlockSpec((1, INTER, DIM), lambda i, j: (j, 0, 0)),
            pl.BlockSpec((1, INTER, DIM), lambda i, j: (j, 0, 0)),
            pl.BlockSpec((1, DIM, INTER), lambda i, j: (j, 0, 0)),
            pl.BlockSpec((1, 1, INTER), lambda i, j: (j, 0, 0)),
            pl.BlockSpec((1, 1, INTER), lambda i, j: (j, 0, 0)),
            pl.BlockSpec((1, 1, DIM), lambda i, j: (j, 0, 0)),
        ],
        out_specs=pl.BlockSpec((BT, DIM), lambda i, j: (i, 0)),
        out_shape=jax.ShapeDtypeStruct((T, DIM), jnp.float32),
        scratch_shapes=[pltpu.VMEM((BT, N_EXPERTS), jnp.float32),
                        pltpu.VMEM((BT, DIM), jnp.bfloat16)],
        compiler_params=pltpu.CompilerParams(
            dimension_semantics=("arbitrary", "arbitrary")),
    )(x, gate_w, w1a, w3a, w2a, b1a, b3a, b2a)


def kernel(x, gate_w, w1, b1, w2, b2, w3, b3, ws1, bs1, ws2, bs2):
    n_shared = ws1.shape[0] // INTER
    bf = jnp.bfloat16
    w1a = jnp.concatenate([w1, ws1.reshape(n_shared, INTER, DIM)],
                          axis=0).astype(bf)
    w3a = jnp.concatenate([w3, jnp.zeros((n_shared, INTER, DIM), w3.dtype)],
                          axis=0).astype(bf)
    w2a = jnp.concatenate(
        [w2, jnp.moveaxis(ws2.reshape(DIM, n_shared, INTER), 1, 0)],
        axis=0).astype(bf)
    b1a = jnp.concatenate([b1, bs1.reshape(n_shared, INTER)], axis=0)
    b3a = jnp.concatenate([b3, jnp.ones((n_shared, INTER), b3.dtype)], axis=0)
    b2a = jnp.concatenate(
        [b2, jnp.stack([bs2, jnp.zeros_like(bs2)], axis=0)], axis=0)
    return _moe_fused(x, gate_w, w1a, w3a, w2a,
                      b1a[:, None, :], b3a[:, None, :], b2a[:, None, :])


# speedup vs baseline: 1.2974x; 1.0710x over previous
import jax
import jax.numpy as jnp
from jax.experimental import pallas as pl
from jax.experimental.pallas import tpu as pltpu

DIM = 1024
INTER = 512
N_EXPERTS = 8
N_LOCAL = 4
N_AUG = 6
T = 2048
BT = 256


def _moe_body(x_ref, gate_ref, w1_ref, w3_ref, w2_ref, b1_ref, b3_ref,
              b2_ref, out_ref, gw_scr, xb_scr):
    j = pl.program_id(1)

    @pl.when(j == 0)
    def _():
        xb32 = x_ref[...]
        xb_scr[...] = xb32.astype(jnp.bfloat16)
        logits = jax.lax.dot_general(
            xb32, gate_ref[...], (((1,), (1,)), ((), ())),
            preferred_element_type=jnp.float32)
        mx = jnp.max(logits, axis=1, keepdims=True)
        ex = jnp.exp(logits - mx)
        scores = ex / jnp.sum(ex, axis=1, keepdims=True)

        iota8 = jax.lax.broadcasted_iota(jnp.int32, (BT, N_EXPERTS), 1)
        m0 = jnp.max(scores, axis=1, keepdims=True)
        idx0 = jnp.min(jnp.where(scores == m0, iota8, N_EXPERTS), axis=1,
                       keepdims=True)
        s1 = jnp.where(iota8 == idx0, -jnp.inf, scores)
        m1 = jnp.max(s1, axis=1, keepdims=True)
        idx1 = jnp.min(jnp.where(s1 == m1, iota8, N_EXPERTS), axis=1,
                       keepdims=True)
        gw_scr[...] = (jnp.where(idx0 == iota8, m0, 0.0)
                       + jnp.where(idx1 == iota8, m1, 0.0))

    xb = xb_scr[...]
    gw_routed = jnp.sum(
        jnp.where(jax.lax.broadcasted_iota(jnp.int32, (BT, N_EXPERTS), 1)
                  == j, gw_scr[...], 0.0),
        axis=1, keepdims=True)
    gw = jnp.where(j < N_LOCAL, gw_routed, 1.0)

    w1 = w1_ref[j]
    w3 = w3_ref[j]
    w2 = w2_ref[j]
    h1 = jax.lax.dot_general(xb, w1, (((1,), (1,)), ((), ())),
                             preferred_element_type=jnp.float32) + b1_ref[j]
    h3 = jax.lax.dot_general(xb, w3, (((1,), (1,)), ((), ())),
                             preferred_element_type=jnp.float32) + b3_ref[j]
    h = (h1 * jax.lax.logistic(h1)) * h3
    o = jax.lax.dot_general(h.astype(jnp.bfloat16), w2,
                            (((1,), (1,)), ((), ())),
                            preferred_element_type=jnp.float32) + b2_ref[j]
    contrib = gw * o

    @pl.when(j == 0)
    def _():
        out_ref[...] = contrib

    @pl.when(j > 0)
    def _():
        out_ref[...] = out_ref[...] + contrib


@jax.jit
def _moe_fused(x, gate_w, w1a, w3a, w2a, b1a, b3a, b2a):
    grid = (T // BT, N_AUG)
    return pl.pallas_call(
        _moe_body,
        grid=grid,
        in_specs=[
            pl.BlockSpec((BT, DIM), lambda i, j: (i, 0)),
            pl.BlockSpec((N_EXPERTS, DIM), lambda i, j: (0, 0)),
            pl.BlockSpec((N_AUG, INTER, DIM), lambda i, j: (0, 0, 0)),
            pl.BlockSpec((N_AUG, INTER, DIM), lambda i, j: (0, 0, 0)),
            pl.BlockSpec((N_AUG, DIM, INTER), lambda i, j: (0, 0, 0)),
            pl.BlockSpec((N_AUG, 1, INTER), lambda i, j: (0, 0, 0)),
            pl.BlockSpec((N_AUG, 1, INTER), lambda i, j: (0, 0, 0)),
            pl.BlockSpec((N_AUG, 1, DIM), lambda i, j: (0, 0, 0)),
        ],
        out_specs=pl.BlockSpec((BT, DIM), lambda i, j: (i, 0)),
        out_shape=jax.ShapeDtypeStruct((T, DIM), jnp.float32),
        scratch_shapes=[pltpu.VMEM((BT, N_EXPERTS), jnp.float32),
                        pltpu.VMEM((BT, DIM), jnp.bfloat16)],
        compiler_params=pltpu.CompilerParams(
            dimension_semantics=("arbitrary", "arbitrary")),
    )(x, gate_w, w1a, w3a, w2a, b1a, b3a, b2a)


def kernel(x, gate_w, w1, b1, w2, b2, w3, b3, ws1, bs1, ws2, bs2):
    n_shared = ws1.shape[0] // INTER
    bf = jnp.bfloat16
    w1a = jnp.concatenate([w1, ws1.reshape(n_shared, INTER, DIM)],
                          axis=0).astype(bf)
    w3a = jnp.concatenate([w3, jnp.zeros((n_shared, INTER, DIM), w3.dtype)],
                          axis=0).astype(bf)
    w2a = jnp.concatenate(
        [w2, jnp.moveaxis(ws2.reshape(DIM, n_shared, INTER), 1, 0)],
        axis=0).astype(bf)
    b1a = jnp.concatenate([b1, bs1.reshape(n_shared, INTER)], axis=0)
    b3a = jnp.concatenate([b3, jnp.ones((n_shared, INTER), b3.dtype)], axis=0)
    b2a = jnp.concatenate(
        [b2, jnp.stack([bs2, jnp.zeros_like(bs2)], axis=0)], axis=0)
    return _moe_fused(x, gate_w, w1a, w3a, w2a,
                      b1a[:, None, :], b3a[:, None, :], b2a[:, None, :])
